# Initial kernel scaffold; baseline (speedup 1.0000x reference)
import jax
import jax.numpy as jnp
from jax.experimental import pallas as pl


def kernel(x, batch, W_xg, b_xg, W_lin):
    raise NotImplementedError("write your pallas kernel here")


# speedup vs baseline: 2.2670x; 2.2670x over previous
import functools

import jax
import jax.numpy as jnp
from jax import lax
from jax.experimental import pallas as pl
from jax.experimental.pallas import tpu as pltpu
from jax.experimental.pallas import tpu_sc as plsc

NUM_GRAPHS = 2048
NUM_NODES = 10000
FEAT = 512

NC = 2
NS = 16
NW = NC * NS
SEG_PER_TILE = NUM_GRAPHS // NW
WIN = 64


def _sc_pool(x, batch_i32, zsums, zcnt):
    mesh = plsc.VectorSubcoreMesh(core_axis_name="c", subcore_axis_name="s")

    @functools.partial(
        pl.kernel,
        out_type=[
            jax.ShapeDtypeStruct((NUM_GRAPHS, FEAT), jnp.float32),
            jax.ShapeDtypeStruct((NUM_GRAPHS, 16), jnp.float32),
        ],
        mesh=mesh,
        compiler_params=pltpu.CompilerParams(needs_layout_passes=False),
        scratch_types=[
            pltpu.VMEM((NUM_NODES,), jnp.int32),
            pltpu.VMEM((WIN, FEAT), jnp.float32),
            pltpu.VMEM((SEG_PER_TILE + 1, FEAT), jnp.float32),
            pltpu.VMEM((SEG_PER_TILE + 1, 16), jnp.float32),
        ],
    )
    def body(x_hbm, b_hbm, zs_hbm, zc_hbm, sums_out, cnt_out,
             idbuf, xbuf, acc, cnt):
        cid = lax.axis_index("c")
        sid = lax.axis_index("s")
        wid = cid * NS + sid
        seg_lo = wid * SEG_PER_TILE

        pltpu.sync_copy(zs_hbm, acc.at[pl.ds(0, SEG_PER_TILE)])
        pltpu.sync_copy(zc_hbm, cnt.at[pl.ds(0, SEG_PER_TILE)])
        pltpu.sync_copy(b_hbm, idbuf)

        def scan_body(j, carry):
            lo_v, hi_v = carry
            v = idbuf[pl.ds(j * 16, 16)]
            lo_v = lo_v + (v < seg_lo).astype(jnp.int32)
            hi_v = hi_v + (v < seg_lo + SEG_PER_TILE).astype(jnp.int32)
            return lo_v, hi_v

        z16 = jnp.zeros((16,), jnp.int32)
        lo_v, hi_v = lax.fori_loop(0, NUM_NODES // 16, scan_body, (z16, z16))
        lo = jnp.sum(lo_v)
        hi = jnp.sum(hi_v)

        w0 = (lo // WIN) * WIN
        nwin = (hi - w0 + (WIN - 1)) // WIN
        iota16 = lax.iota(jnp.int32, 16)
        e0_16 = jnp.where(iota16 == 0, 1.0, 0.0).astype(jnp.float32)

        def win_body(w, carry):
            s_true = w0 + WIN * w
            s = jnp.minimum(s_true, NUM_NODES - WIN)
            pltpu.sync_copy(x_hbm.at[pl.ds(s, WIN)], xbuf)
            lo2 = jnp.maximum(lo, s_true)

            def grp_body(k, carry2):
                g0 = s + k * 16
                idv = idbuf[pl.ds(g0, 16)]
                for j in range(16):
                    g = g0 + j
                    valid = (g >= lo2) & (g < hi)
                    sid_t = jnp.where(valid, idv[j] - seg_lo, SEG_PER_TILE)
                    rloc = k * 16 + j
                    plsc.addupdate(cnt.at[sid_t], e0_16)
                    for c in range(FEAT // 16):
                        plsc.addupdate(acc.at[sid_t, pl.ds(c * 16, 16)],
                                       xbuf[rloc, pl.ds(c * 16, 16)])
                return carry2

            lax.fori_loop(0, WIN // 16, grp_body, 0)
            return carry

        lax.fori_loop(0, nwin, win_body, 0)

        pltpu.sync_copy(acc.at[pl.ds(0, SEG_PER_TILE)],
                        sums_out.at[pl.ds(seg_lo, SEG_PER_TILE)])
        pltpu.sync_copy(cnt.at[pl.ds(0, SEG_PER_TILE)],
                        cnt_out.at[pl.ds(seg_lo, SEG_PER_TILE)])

    return body(x, batch_i32, zsums, zcnt)


def _tc_body(s_ref, c_ref, wxg_ref, b_ref, wlin_ref, o_ref):
    denom = jnp.maximum(jnp.sum(c_ref[...], axis=1, keepdims=True), 1.0)
    m = s_ref[...] / denom
    h = lax.dot_general(m, wxg_ref[...], (((1,), (1,)), ((), ())),
                        preferred_element_type=jnp.float32) + b_ref[...]
    r = jnp.maximum(h, 0.0)
    o_ref[...] = h + lax.dot_general(r, wlin_ref[...], (((1,), (1,)), ((), ())),
                                     preferred_element_type=jnp.float32)


def _tc_dense(sums, counts, W_xg, b_xg2, W_lin):
    blk = 256
    grid = NUM_GRAPHS // blk
    return pl.pallas_call(
        _tc_body,
        grid=(grid,),
        in_specs=[
            pl.BlockSpec((blk, FEAT), lambda i: (i, 0)),
            pl.BlockSpec((blk, 16), lambda i: (i, 0)),
            pl.BlockSpec((FEAT, FEAT), lambda i: (0, 0)),
            pl.BlockSpec((1, FEAT), lambda i: (0, 0)),
            pl.BlockSpec((FEAT, FEAT), lambda i: (0, 0)),
        ],
        out_specs=pl.BlockSpec((blk, FEAT), lambda i: (i, 0)),
        out_shape=jax.ShapeDtypeStruct((NUM_GRAPHS, FEAT), jnp.float32),
    )(sums, counts, W_xg, b_xg2, W_lin)


@jax.jit
def kernel(x, batch, W_xg, b_xg, W_lin):
    batch_i32 = batch.astype(jnp.int32)
    zsums = jnp.zeros((SEG_PER_TILE, FEAT), jnp.float32)
    zcnt = jnp.zeros((SEG_PER_TILE, 16), jnp.float32)
    sums, counts = _sc_pool(x, batch_i32, zsums, zcnt)
    return _tc_dense(sums, counts, W_xg, b_xg.reshape(1, FEAT), W_lin)
